# baseline (speedup 1.0000x reference)
import jax
import jax.numpy as jnp
from jax import lax
from jax.experimental import pallas as pl
from jax.experimental.pallas import tpu as pltpu
from jax.experimental.pallas import tpu_sc as plsc

NC = 2
NS = 16
NW = NC * NS

VOCAB = 1000000
MAXLEN = 200
EMBED = 64
BATCH = 4096
SEQ = 200
SPLIT = 128
REST = SEQ - SPLIT

K = 2
B_PER_W = BATCH // NW
N_CHUNKS = B_PER_W // K
VREGS = EMBED // 16

_MESH = plsc.VectorSubcoreMesh(core_axis_name="c", subcore_axis_name="s")


def _wid():
    return lax.axis_index("s") * NC + lax.axis_index("c")


def _emb_body(xa_hbm, xb_hbm, tok_hbm, pos_hbm, out_hbm,
              pos_v, idxa_a, idxb_a, idxa_b, idxb_b, rows_a, rows_b,
              isem_a, isem_b, gsem_a, gsem_b, outsem):
    base_b = _wid() * B_PER_W

    pltpu.sync_copy(pos_hbm, pos_v)

    def prefetch(g, idxa, idxb, isem):
        b0 = base_b + g * K
        pltpu.async_copy(xa_hbm.at[pl.ds(b0, K)], idxa, isem)
        pltpu.async_copy(xb_hbm.at[pl.ds(b0, K)], idxb, isem)

    def launch(g, idxa, idxb, rows, isem, gsem):
        pltpu.make_async_copy(xa_hbm.at[pl.ds(0, K)], idxa, isem).wait()
        pltpu.make_async_copy(xb_hbm.at[pl.ds(0, K)], idxb, isem).wait()
        for k in range(K):
            pltpu.async_copy(
                tok_hbm.at[idxa.at[k]], rows.at[k, pl.ds(0, SPLIT)], gsem
            )
            pltpu.async_copy(
                tok_hbm.at[idxb.at[k]], rows.at[k, pl.ds(SPLIT, REST)], gsem
            )

    def drain(rows, gsem):
        for k in range(K):
            pltpu.make_async_copy(
                tok_hbm.at[pl.ds(0, SPLIT)], rows.at[k, pl.ds(0, SPLIT)], gsem
            ).wait()
            pltpu.make_async_copy(
                tok_hbm.at[pl.ds(0, REST)],
                rows.at[k, pl.ds(SPLIT, REST)],
                gsem,
            ).wait()

    def add(rows):
        def body(s):
            for d in range(VREGS):
                pv = pos_v[s, pl.ds(d * 16, 16)]
                for k in range(K):
                    rows[k, s, pl.ds(d * 16, 16)] = (
                        rows[k, s, pl.ds(d * 16, 16)] + pv
                    )
        plsc.parallel_loop(0, SEQ, unroll=2)(body)

    def put(g, rows):
        b0 = base_b + g * K
        pltpu.async_copy(
            rows.at[:, :, pl.ds(0, EMBED)], out_hbm.at[pl.ds(b0, K)], outsem
        )

    def wait_out():
        pltpu.make_async_copy(
            rows_a.at[:, :, pl.ds(0, EMBED)],
            out_hbm.at[pl.ds(0, K)],
            outsem,
        ).wait()

    prefetch(0, idxa_a, idxb_a, isem_a)
    prefetch(1, idxa_b, idxb_b, isem_b)
    launch(0, idxa_a, idxb_a, rows_a, isem_a, gsem_a)
    launch(1, idxa_b, idxb_b, rows_b, isem_b, gsem_b)
    drain(rows_a, gsem_a)
    prefetch(2, idxa_a, idxb_a, isem_a)
    add(rows_a)
    put(0, rows_a)

    def body(g2, c):
        g = 1 + 2 * g2
        wait_out()
        launch(g + 1, idxa_a, idxb_a, rows_a, isem_a, gsem_a)
        drain(rows_b, gsem_b)
        prefetch(g + 2, idxa_b, idxb_b, isem_b)
        add(rows_b)
        put(g, rows_b)
        wait_out()
        launch(g + 2, idxa_b, idxb_b, rows_b, isem_b, gsem_b)
        drain(rows_a, gsem_a)
        prefetch(jnp.minimum(g + 3, N_CHUNKS - 1), idxa_a, idxb_a, isem_a)
        add(rows_a)
        put(g + 1, rows_a)
        return c

    lax.fori_loop(0, (N_CHUNKS - 2) // 2, body, 0)

    drain(rows_b, gsem_b)
    add(rows_b)
    put(N_CHUNKS - 1, rows_b)
    pltpu.make_async_copy(xa_hbm.at[pl.ds(0, K)], idxa_a, isem_a).wait()
    pltpu.make_async_copy(xb_hbm.at[pl.ds(0, K)], idxb_a, isem_a).wait()
    wait_out()
    wait_out()


@jax.jit
def _emb(xa, xb, token_table, pos_table):
    k1 = pl.kernel(
        _emb_body,
        out_type=jax.ShapeDtypeStruct((BATCH, SEQ, EMBED), jnp.float32),
        mesh=_MESH,
        scratch_types=[
            pltpu.VMEM((MAXLEN, EMBED), jnp.float32),
            pltpu.VMEM((K, SPLIT), jnp.int32),
            pltpu.VMEM((K, REST), jnp.int32),
            pltpu.VMEM((K, SPLIT), jnp.int32),
            pltpu.VMEM((K, REST), jnp.int32),
            pltpu.VMEM((K, SEQ, 128), jnp.float32),
            pltpu.VMEM((K, SEQ, 128), jnp.float32),
            pltpu.SemaphoreType.DMA,
            pltpu.SemaphoreType.DMA,
            pltpu.SemaphoreType.DMA,
            pltpu.SemaphoreType.DMA,
            pltpu.SemaphoreType.DMA,
        ],
        compiler_params=pltpu.CompilerParams(use_tc_tiling_on_sc=False),
    )
    return k1(xa, xb, token_table, pos_table)


def kernel(x, token_table, pos_table):
    x = x.astype(jnp.int32)
    tabp = jnp.pad(token_table, ((0, 0), (0, 128 - EMBED)))
    return _emb(x[:, :SPLIT], x[:, SPLIT:], tabp, pos_table)


# speedup vs baseline: 1.0173x; 1.0173x over previous
import jax
import jax.numpy as jnp
from jax import lax
from jax.experimental import pallas as pl
from jax.experimental.pallas import tpu as pltpu
from jax.experimental.pallas import tpu_sc as plsc

NC = 2
NS = 16
NW = NC * NS

VOCAB = 1000000
MAXLEN = 200
EMBED = 64
BATCH = 4096
SEQ = 200
SPLIT = 128
REST = SEQ - SPLIT

K = 4
B_PER_W = BATCH // NW
N_CHUNKS = B_PER_W // K
VREGS = EMBED // 16

_MESH = plsc.VectorSubcoreMesh(core_axis_name="c", subcore_axis_name="s")


def _wid():
    return lax.axis_index("s") * NC + lax.axis_index("c")


def _emb_body(xa_hbm, xb_hbm, tok_hbm, pos_hbm, out_hbm,
              pos_v, idxa_a, idxb_a, idxa_b, idxb_b, rows_a, rows_b,
              isem_a, isem_b, gsem_a, gsem_b, outsem):
    base_b = _wid() * B_PER_W

    pltpu.sync_copy(pos_hbm, pos_v)

    def prefetch(g, idxa, idxb, isem):
        b0 = base_b + g * K
        pltpu.async_copy(xa_hbm.at[pl.ds(b0, K)], idxa, isem)
        pltpu.async_copy(xb_hbm.at[pl.ds(b0, K)], idxb, isem)

    def launch(g, idxa, idxb, rows, isem, gsem):
        pltpu.make_async_copy(xa_hbm.at[pl.ds(0, K)], idxa, isem).wait()
        pltpu.make_async_copy(xb_hbm.at[pl.ds(0, K)], idxb, isem).wait()
        for k in range(K):
            pltpu.async_copy(
                tok_hbm.at[idxa.at[k]], rows.at[k, pl.ds(0, SPLIT)], gsem
            )
            pltpu.async_copy(
                tok_hbm.at[idxb.at[k]], rows.at[k, pl.ds(SPLIT, REST)], gsem
            )

    def drain(rows, gsem):
        pltpu.make_async_copy(out_hbm.at[pl.ds(0, K)], rows, gsem).wait()

    def add(rows):
        def body(s):
            for d in range(VREGS):
                pv = pos_v[s, pl.ds(d * 16, 16)]
                for k in range(K):
                    rows[k, s, pl.ds(d * 16, 16)] = (
                        rows[k, s, pl.ds(d * 16, 16)] + pv
                    )
        plsc.parallel_loop(0, SEQ, unroll=2)(body)

    def put(g, rows):
        b0 = base_b + g * K
        pltpu.async_copy(rows, out_hbm.at[pl.ds(b0, K)], outsem)

    def wait_out():
        pltpu.make_async_copy(rows_a, out_hbm.at[pl.ds(0, K)], outsem).wait()

    prefetch(0, idxa_a, idxb_a, isem_a)
    prefetch(1, idxa_b, idxb_b, isem_b)
    launch(0, idxa_a, idxb_a, rows_a, isem_a, gsem_a)
    launch(1, idxa_b, idxb_b, rows_b, isem_b, gsem_b)
    drain(rows_a, gsem_a)
    prefetch(2, idxa_a, idxb_a, isem_a)
    add(rows_a)
    put(0, rows_a)

    def body(g2, c):
        g = 1 + 2 * g2
        wait_out()
        launch(g + 1, idxa_a, idxb_a, rows_a, isem_a, gsem_a)
        drain(rows_b, gsem_b)
        prefetch(g + 2, idxa_b, idxb_b, isem_b)
        add(rows_b)
        put(g, rows_b)
        wait_out()
        launch(g + 2, idxa_b, idxb_b, rows_b, isem_b, gsem_b)
        drain(rows_a, gsem_a)
        prefetch(jnp.minimum(g + 3, N_CHUNKS - 1), idxa_a, idxb_a, isem_a)
        add(rows_a)
        put(g + 1, rows_a)
        return c

    lax.fori_loop(0, (N_CHUNKS - 2) // 2, body, 0)

    drain(rows_b, gsem_b)
    add(rows_b)
    put(N_CHUNKS - 1, rows_b)
    pltpu.make_async_copy(xa_hbm.at[pl.ds(0, K)], idxa_a, isem_a).wait()
    pltpu.make_async_copy(xb_hbm.at[pl.ds(0, K)], idxb_a, isem_a).wait()
    wait_out()
    wait_out()


@jax.jit
def _emb(xa, xb, token_table, pos_table):
    k1 = pl.kernel(
        _emb_body,
        out_type=jax.ShapeDtypeStruct((BATCH, SEQ, EMBED), jnp.float32),
        mesh=_MESH,
        scratch_types=[
            pltpu.VMEM((MAXLEN, EMBED), jnp.float32),
            pltpu.VMEM((K, SPLIT), jnp.int32),
            pltpu.VMEM((K, REST), jnp.int32),
            pltpu.VMEM((K, SPLIT), jnp.int32),
            pltpu.VMEM((K, REST), jnp.int32),
            pltpu.VMEM((K, SEQ, EMBED), jnp.float32),
            pltpu.VMEM((K, SEQ, EMBED), jnp.float32),
            pltpu.SemaphoreType.DMA,
            pltpu.SemaphoreType.DMA,
            pltpu.SemaphoreType.DMA,
            pltpu.SemaphoreType.DMA,
            pltpu.SemaphoreType.DMA,
        ],
        compiler_params=pltpu.CompilerParams(use_tc_tiling_on_sc=False),
    )
    return k1(xa, xb, token_table, pos_table)


def kernel(x, token_table, pos_table):
    x = x.astype(jnp.int32)
    return _emb(x[:, :SPLIT], x[:, SPLIT:], token_table, pos_table)
